# baseline (speedup 1.0000x reference)
import functools

import jax
import jax.numpy as jnp
from jax import lax
from jax.experimental import pallas as pl
from jax.experimental.pallas import tpu as pltpu
from jax.experimental.pallas import tpu_sc as plsc

EMB = 64
EPS = 1e-5
CH = 128
NW = 32
NT = 16
BLK_N = 2000
BLK_E = 2048
F32 = jnp.float32


def _dot(x, w):
    return jnp.dot(x, w, preferred_element_type=F32)


def _embed_body(x_ref, g_ref, b_ref, w1_ref, b1_ref, w2_ref, b2_ref, o_ref):
    x = x_ref[...]
    m = jnp.mean(x, axis=-1, keepdims=True)
    v = jnp.mean((x - m) ** 2, axis=-1, keepdims=True)
    xn = (x - m) * lax.rsqrt(v + EPS) * g_ref[...] + b_ref[...]
    h1 = jnp.maximum(_dot(xn, w1_ref[...].T) + b1_ref[...], 0.0)
    o_ref[...] = jnp.maximum(_dot(h1, w2_ref[...].T) + b2_ref[...], 0.0)


def _embed_call(x, g, b, w1, b1, w2, b2):
    n, f = x.shape
    full = lambda s: pl.BlockSpec(s, lambda i: (0, 0))
    return pl.pallas_call(
        _embed_body,
        grid=(n // BLK_N,),
        in_specs=[pl.BlockSpec((BLK_N, f), lambda i: (i, 0)),
                  full((1, f)), full((1, f)), full((EMB, f)),
                  full((1, EMB)), full((EMB, EMB)), full((1, EMB))],
        out_specs=pl.BlockSpec((BLK_N, EMB), lambda i: (i, 0)),
        out_shape=jax.ShapeDtypeStruct((n, EMB), F32),
    )(x, g.reshape(1, f), b.reshape(1, f), w1,
      b1.reshape(1, EMB), w2, b2.reshape(1, EMB))


def _pre_body(r_ref, l_ref, lw_ref, bias_ref, rw_ref, t_ref, tr_ref):
    a = _dot(r_ref[...], lw_ref[...].T) + bias_ref[...]
    b = _dot(l_ref[...], rw_ref[...].T)
    t_ref[...] = jnp.concatenate([a, b], axis=-1)
    tr_ref[...] = jnp.concatenate([b, a], axis=-1)


def _pre_call(right, left, left_w, bias_eff, right_w):
    n = right.shape[0]
    full = lambda s: pl.BlockSpec(s, lambda i: (0, 0))
    blk = pl.BlockSpec((BLK_N, EMB), lambda i: (i, 0))
    blk2 = pl.BlockSpec((BLK_N, 2 * EMB), lambda i: (i, 0))
    return pl.pallas_call(
        _pre_body,
        grid=(n // BLK_N,),
        in_specs=[blk, blk, full((EMB, EMB)), full((1, EMB)), full((EMB, EMB))],
        out_specs=[blk2, blk2],
        out_shape=[jax.ShapeDtypeStruct((n, 2 * EMB), F32),
                   jax.ShapeDtypeStruct((n, 2 * EMB), F32)],
    )(right, left, left_w, bias_eff.reshape(1, EMB), right_w)


def _act_body(s_ref, g_ref, b_ref, h_ref):
    s = s_ref[...]
    m = jnp.mean(s, axis=-1, keepdims=True)
    v = jnp.mean((s - m) ** 2, axis=-1, keepdims=True)
    h = jnp.maximum((s - m) * lax.rsqrt(v + EPS) * g_ref[...] + b_ref[...], 0.0)
    q = EMB // 4
    h_ref[...] = jnp.concatenate(
        [h[:, k * q:(k + 1) * q] for k in range(4)], axis=0)


def _act_call(s, g, b):
    e = s.shape[0]
    q = EMB // 4
    full = lambda sh: pl.BlockSpec(sh, lambda i: (0, 0))
    return pl.pallas_call(
        _act_body,
        grid=(e // BLK_E,),
        in_specs=[pl.BlockSpec((BLK_E, EMB), lambda i: (i, 0)),
                  full((1, EMB)), full((1, EMB))],
        out_specs=pl.BlockSpec((4 * BLK_E, q), lambda i: (i, 0)),
        out_shape=jax.ShapeDtypeStruct((4 * e, q), F32),
    )(s, g.reshape(1, EMB), b.reshape(1, EMB))


def _post_body(a0_ref, a1_ref, a2_ref, a3_ref, cnt_ref, r_ref,
               fw0_ref, fw1_ref, fw2_ref, fw3_ref, fb_ref,
               pg_ref, pb_ref, w1a_ref, w1b_ref, b1_ref, w2_ref, b2_ref, o_ref):
    y = (_dot(a0_ref[...], fw0_ref[...]) + _dot(a1_ref[...], fw1_ref[...])
         + _dot(a2_ref[...], fw2_ref[...]) + _dot(a3_ref[...], fw3_ref[...])
         + cnt_ref[...] * fb_ref[...])
    m = jnp.mean(y, axis=-1, keepdims=True)
    v = jnp.mean((y - m) ** 2, axis=-1, keepdims=True)
    post = (y - m) * lax.rsqrt(v + EPS) * pg_ref[...] + pb_ref[...]
    o = jnp.maximum(_dot(post, w1a_ref[...].T) + _dot(r_ref[...], w1b_ref[...].T)
                    + b1_ref[...], 0.0)
    o_ref[...] = _dot(o, w2_ref[...].T) + b2_ref[...]


def _post_call(aggs, cnt, right, p):
    n = right.shape[0]
    q = EMB // 4
    full = lambda s: pl.BlockSpec(s, lambda i: (0, 0))
    blk = pl.BlockSpec((BLK_N, EMB), lambda i: (i, 0))
    blkq = pl.BlockSpec((BLK_N, q), lambda i: (i, 0))
    fwt = p['fin_W'].T
    return pl.pallas_call(
        _post_body,
        grid=(n // BLK_N,),
        in_specs=[blkq, blkq, blkq, blkq,
                  pl.BlockSpec((BLK_N, 1), lambda i: (i, 0)), blk,
                  full((q, EMB)), full((q, EMB)), full((q, EMB)), full((q, EMB)),
                  full((1, EMB)),
                  full((1, EMB)), full((1, EMB)),
                  full((EMB, EMB)), full((EMB, EMB)), full((1, EMB)),
                  full((EMB, EMB)), full((1, EMB))],
        out_specs=blk,
        out_shape=jax.ShapeDtypeStruct((n, EMB), F32),
    )(*aggs, cnt, right,
      fwt[:q], fwt[q:2 * q], fwt[2 * q:3 * q], fwt[3 * q:],
      p['fin_b'].reshape(1, EMB),
      p['post_ln_g'].reshape(1, EMB), p['post_ln_b'].reshape(1, EMB),
      p['out1_W'][:, :EMB], p['out1_W'][:, EMB:], p['out1_b'].reshape(1, EMB),
      p['out2_W'], p['out2_b'].reshape(1, EMB))


def _head_body(v_ref, w1_ref, b1_ref, w2_ref, o_ref):
    o = jnp.maximum(_dot(v_ref[...], w1_ref[...].T) + b1_ref[...], 0.0)
    o_ref[...] = _dot(o, w2_ref[...].T)


def _head_call(v, w1, b1, w2):
    n = v.shape[0]
    full = lambda s: pl.BlockSpec(s, lambda i: (0, 0))
    return pl.pallas_call(
        _head_body,
        grid=(n // BLK_N,),
        in_specs=[pl.BlockSpec((BLK_N, EMB), lambda i: (i, 0)),
                  full((EMB, EMB)), full((1, EMB)), full((1, EMB))],
        out_specs=pl.BlockSpec((BLK_N, 1), lambda i: (i, 0)),
        out_shape=jax.ShapeDtypeStruct((n, 1), F32),
    )(v, w1, b1.reshape(1, EMB), w2)


def _gather_call(a, b, dst2d, src2d):
    n_chunks = dst2d.shape[0]
    n_per = n_chunks // NW
    e_pad = n_chunks * CH
    mesh = plsc.VectorSubcoreMesh(core_axis_name="c", subcore_axis_name="s")

    @functools.partial(
        pl.kernel,
        out_type=jax.ShapeDtypeStruct((e_pad, EMB), F32),
        mesh=mesh,
        scratch_types=[pltpu.VMEM((n_per, CH), jnp.int32),
                       pltpu.VMEM((n_per, CH), jnp.int32),
                       pltpu.VMEM((CH, 2 * EMB), F32),
                       pltpu.VMEM((CH, 2 * EMB), F32),
                       pltpu.VMEM((CH, EMB), F32)],
    )
    def gather_kernel(a_hbm, b_hbm, dst_hbm, src_hbm, out_hbm,
                      idx_d, idx_s, rows_a, rows_b, sbuf):
        wid = lax.axis_index("s") * 2 + lax.axis_index("c")
        c0 = wid * n_per
        pltpu.sync_copy(dst_hbm.at[pl.ds(c0, n_per)], idx_d)
        pltpu.sync_copy(src_hbm.at[pl.ds(c0, n_per)], idx_s)

        def body(g, carry):
            pltpu.sync_copy(a_hbm.at[idx_d.at[g]], rows_a)
            pltpu.sync_copy(b_hbm.at[idx_s.at[g]], rows_b)

            def add_row(i, carry2):
                for j in range(EMB // 16):
                    sbuf[i, pl.ds(16 * j, 16)] = (
                        rows_a[i, pl.ds(16 * j, 16)]
                        + rows_b[i, pl.ds(EMB + 16 * j, 16)])
                return carry2

            lax.fori_loop(0, CH, add_row, 0)
            pltpu.sync_copy(sbuf, out_hbm.at[pl.ds((c0 + g) * CH, CH)])
            return carry

        lax.fori_loop(0, n_per, body, 0)

    return gather_kernel(a, b, dst2d, src2d)


def _scatter_call(h_il, dst2d, n_pad):
    n_chunks = dst2d.shape[0]
    n_per = n_chunks // NT
    q = EMB // 4
    cpb = BLK_E // CH
    half = n_pad // 2
    acc_r = -(-(half + 1) // (128 * NT)) * (128 * NT)
    rpt = acc_r // NT
    nz = rpt // 128
    mesh = plsc.VectorSubcoreMesh(core_axis_name="c", subcore_axis_name="s")
    assert acc_r - half >= 64

    @functools.partial(
        pl.kernel,
        out_type=jax.ShapeDtypeStruct((8 * acc_r, q), F32),
        mesh=mesh,
        scratch_types=[pltpu.VMEM((n_per, CH), jnp.int32),
                       pltpu.VMEM((CH,), jnp.int32),
                       pltpu.VMEM((CH, q), F32),
                       pltpu.VMEM((CH, q), F32),
                       pltpu.VMEM_SHARED((acc_r, q), F32)],
    )
    def scatter_kernel(h_hbm, dst_hbm, out_hbm, idx, idx_c, upd, stg, acc):
        c = lax.axis_index("c")
        s = lax.axis_index("s")
        r0 = s * rpt
        zvec = jnp.zeros((16,), F32)
        iv16 = lax.iota(jnp.int32, 16)
        pltpu.sync_copy(dst_hbm.at[pl.ds(s * n_per, n_per)], idx)

        def zfill(i, carry2):
            stg[i, pl.ds(0, 16)] = zvec
            return carry2

        lax.fori_loop(0, CH, zfill, 0)

        def phase(ph, carry):
            qq = 2 * c + (ph % 2)
            lo = (ph // 2) * half

            def zcopy(k, carry2):
                base = r0 + k * CH
                for j in range(CH // 16):
                    idx_c[pl.ds(16 * j, 16)] = base + 16 * j + iv16
                pltpu.sync_copy(stg, acc.at[idx_c])
                return carry2

            lax.fori_loop(0, nz, zcopy, 0)
            plsc.subcore_barrier()

            def body(g, carry2):
                gg = s * n_per + g
                row0 = ((gg // cpb) * 4 + qq) * BLK_E + (gg % cpb) * CH
                pltpu.sync_copy(h_hbm.at[pl.ds(row0, CH)], upd)
                for j in range(CH // 16):
                    v = idx[g, pl.ds(16 * j, 16)]
                    vr = v - lo
                    ok = (vr >= 0) & (vr < half)
                    idx_c[pl.ds(16 * j, 16)] = jnp.where(
                        ok, vr, half + (v & 63))
                pltpu.sync_copy(upd, acc.at[idx_c], add=True)
                return carry2

            lax.fori_loop(0, n_per, body, 0)
            plsc.subcore_barrier()

            def wcopy(k, carry2):
                rk = r0 + k * CH
                for j in range(CH // 16):
                    idx_c[pl.ds(16 * j, 16)] = rk + 16 * j + iv16
                pltpu.sync_copy(acc.at[idx_c], upd)
                pltpu.sync_copy(
                    upd, out_hbm.at[pl.ds((qq * 2 + ph // 2) * acc_r + rk,
                                          CH)])
                return carry2

            lax.fori_loop(0, nz, wcopy, 0)
            plsc.subcore_barrier()
            return carry

        lax.fori_loop(0, 4, phase, 0)

    return scatter_kernel(h_il, dst2d)


def kernel(constraint_features, edge_indices, edge_features, variable_features,
           params):
    del edge_features
    p = params
    n = constraint_features.shape[0]
    e = edge_indices.shape[1]
    e_pad = -(-e // (CH * NW * 8)) * (CH * NW * 8)
    n_pad = -(-(n + 1) // 128) * 128

    ei0 = edge_indices[0]
    ei1 = edge_indices[1]
    pad = e_pad - e
    padg = (jnp.arange(pad, dtype=jnp.int32) % n)
    padt = n + (jnp.arange(pad, dtype=jnp.int32) % NT)

    def packg(ix):
        return jnp.concatenate([ix, padg]).reshape(-1, CH)

    def packt(ix):
        return jnp.concatenate([ix, padt]).reshape(-1, CH)

    dst_g = (packg(ei0), packg(ei1))
    dst_s = (packt(ei0), packt(ei1))
    src_g = (packg(ei1), packg(ei0))

    ones = jnp.ones((4 * e_pad, EMB // 4), F32)
    half = n_pad // 2
    acc_r = -(-(half + 1) // (128 * 16)) * (128 * 16)

    def unpack_quarters(out_cat):
        return [jnp.concatenate(
            [out_cat[(2 * k) * acc_r:(2 * k) * acc_r + half],
             out_cat[(2 * k + 1) * acc_r:(2 * k + 1) * acc_r + half]],
            axis=0)[:n] for k in range(4)]

    cnts = []
    for d in range(2):
        cq = unpack_quarters(_scatter_call(ones, dst_s[d], n_pad))
        cnts.append(cq[0][:, 0:1])

    c = _embed_call(constraint_features, p['cons_ln_g'], p['cons_ln_b'],
                    p['cons_W1'], p['cons_b1'], p['cons_W2'], p['cons_b2'])
    v = _embed_call(variable_features, p['var_ln_g'], p['var_ln_b'],
                    p['var_W1'], p['var_b1'], p['var_W2'], p['var_b2'])

    def conv(cp, left, right, d):
        bias_eff = cp['left_b'] + p['edge_ln_b'][0] * cp['edge_W'][:, 0]
        t, tr = _pre_call(right, left, cp['left_W'], bias_eff, cp['right_W'])
        s = _gather_call(t, tr, dst_g[d], src_g[d])
        h_il = _act_call(s, cp['fin_ln_g'], cp['fin_ln_b'])
        aggs = unpack_quarters(_scatter_call(h_il, dst_s[d], n_pad))
        return _post_call(aggs, cnts[d], right, cp)

    c = conv(p['conv_vc'], v, c, 0)
    v = conv(p['conv_cv'], c, v, 1)
    c = conv(p['conv_vc2'], v, c, 0)
    v = conv(p['conv_cv2'], c, v, 1)

    out = _head_call(v, p['out_W1'], p['out_b1'], p['out_W2'])
    return out[:, 0]
